# baseline (speedup 1.0000x reference)
import functools
import jax
import jax.numpy as jnp
from jax import lax
from jax.experimental import pallas as pl
from jax.experimental.pallas import tpu as pltpu
from jax.experimental.pallas import tpu_sc as plsc

_RS = 16
_D = 768
_L = 16


def _sc_add(in_flat, pos_table):
    n_rows, d = in_flat.shape
    seq = pos_table.shape[0]
    batch = n_rows // seq
    n_workers = 32
    seq_per_w = seq // n_workers
    n_steps = seq_per_w // _RS
    mesh = plsc.VectorSubcoreMesh(core_axis_name="c", subcore_axis_name="s")

    @functools.partial(
        pl.kernel,
        mesh=mesh,
        out_type=jax.ShapeDtypeStruct((n_rows, d), jnp.float32),
        scratch_types=[
            pltpu.VMEM((batch, 2, _RS, _D), jnp.float32),
            pltpu.VMEM((2, _RS, _D), jnp.float32),
            pltpu.SemaphoreType.DMA((batch,)),
            pltpu.SemaphoreType.DMA((batch,)),
            pltpu.SemaphoreType.DMA((2,)),
        ],
    )
    def k(in_hbm, pos_hbm, out_hbm, ibuf, posv, isems, osems, psems):
        wid = lax.axis_index("s") * 2 + lax.axis_index("c")
        seq0 = wid * seq_per_w

        def in_cp(t, b):
            row0 = b * seq + seq0 + t * _RS
            return pltpu.make_async_copy(
                in_hbm.at[pl.ds(row0, _RS)], ibuf.at[b, lax.rem(t, 2)],
                isems.at[b]
            )

        def out_cp(t, b):
            row0 = b * seq + seq0 + t * _RS
            return pltpu.make_async_copy(
                ibuf.at[b, lax.rem(t, 2)], out_hbm.at[pl.ds(row0, _RS)],
                osems.at[b]
            )

        def pos_cp(t):
            return pltpu.make_async_copy(
                pos_hbm.at[pl.ds(seq0 + t * _RS, _RS)],
                posv.at[lax.rem(t, 2)],
                psems.at[lax.rem(t, 2)],
            )

        pos_cp(0).start()
        pos_cp(1).start()
        for b in range(batch):
            in_cp(0, b).start()

        def step(t, _):
            prm = lax.rem(t, 2)
            for b in range(batch):
                @pl.when(t > 0)
                def _():
                    out_cp(t - 1, b).wait()

                @pl.when(t < n_steps - 1)
                def _():
                    in_cp(t + 1, b).start()

                in_cp(t, b).wait()
                if b == 0:
                    pos_cp(t).wait()

                def add_row(r, _):
                    for c in range(_D // _L):
                        sl = pl.ds(c * _L, _L)
                        plsc.addupdate(ibuf.at[b, prm, r, sl], posv[prm, r, sl])
                    return ()

                lax.fori_loop(0, _RS, add_row, ())
                out_cp(t, b).start()
                if b == batch - 1:
                    @pl.when(t + 2 < n_steps)
                    def _():
                        pos_cp(t + 2).start()
            return ()

        lax.fori_loop(0, n_steps, step, ())
        for b in range(batch):
            out_cp(n_steps - 1, b).wait()

    return k(in_flat, pos_table)


def kernel(input_embeds, pos_table):
    b, s, d = input_embeds.shape
    out = _sc_add(input_embeds.reshape(b * s, d), pos_table)
    return out.reshape(b, s, d)
